# baseline (speedup 1.0000x reference)
import functools
import jax
import jax.numpy as jnp
from jax import lax
from jax.experimental import pallas as pl
from jax.experimental.pallas import tpu as pltpu
from jax.experimental.pallas import tpu_sc as plsc

N = 10000
E = 320000
D = 128
R = 3
NTYPES = 4

NC = 2
NS = 16
NW = NC * NS
EPW = E // NW
CH = 128
NCH = 80
EPWP = NCH * CH
SCH = 8
NSUP = NCH // SCH
NPAD = N + 8


def _worker():
    cid = lax.axis_index("c")
    sid = lax.axis_index("s")
    return cid, sid, sid * NC + cid


def _sc_deg_body(dst_hbm, znd_hbm, deg_hbm, dst_sup, ones_v, deg_sh, sem):
    cid, sid, wid = _worker()

    def fill(i, _):
        for j in range(D // 16):
            ones_v[i, pl.ds(j * 16, 16)] = jnp.full((16,), 1.0, jnp.float32)
        return 0
    lax.fori_loop(0, CH, fill, 0)

    @pl.when(sid == 0)
    def _():
        pltpu.sync_copy(znd_hbm, deg_sh)
    plsc.subcore_barrier()

    def sup(s, _):
        pltpu.sync_copy(dst_hbm.at[wid].at[pl.ds(s * SCH, SCH)], dst_sup)
        for j in range(SCH):
            pltpu.async_copy(ones_v, deg_sh.at[dst_sup.at[j]], sem, add=True)
        for j in range(SCH):
            pltpu.make_async_copy(ones_v, deg_sh.at[dst_sup.at[0]],
                                  sem).wait()
        return 0
    lax.fori_loop(0, NSUP, sup, 0)
    plsc.subcore_barrier()

    @pl.when(sid == 0)
    def _():
        pltpu.sync_copy(deg_sh, deg_hbm.at[cid])


def _sc_seg_body(y_hbm, g_hbm, dst_hbm, znd_hbm, acc_hbm,
                 g_sup, dst_sup, rows0, rows1, acc_sh, sem0, sem1):
    cid, sid, wid = _worker()

    @pl.when(sid == 0)
    def _():
        pltpu.sync_copy(znd_hbm, acc_sh)
    plsc.subcore_barrier()

    rows = (rows0, rows1)
    sems = (sem0, sem1)

    def sup(s, _):
        pltpu.sync_copy(g_hbm.at[wid].at[pl.ds(s * SCH, SCH)], g_sup)
        pltpu.sync_copy(dst_hbm.at[wid].at[pl.ds(s * SCH, SCH)], dst_sup)
        pltpu.async_copy(y_hbm.at[g_sup.at[0]], rows0, sem0)
        for j in range(SCH):
            b = j % 2
            pltpu.make_async_copy(y_hbm.at[g_sup.at[0]], rows[b],
                                  sems[b]).wait()
            if j + 1 < SCH:
                pltpu.async_copy(y_hbm.at[g_sup.at[j + 1]], rows[1 - b],
                                 sems[1 - b])
            pltpu.sync_copy(rows[b], acc_sh.at[dst_sup.at[j]], add=True)
        return 0
    lax.fori_loop(0, NSUP, sup, 0)
    plsc.subcore_barrier()

    @pl.when(sid == 0)
    def _():
        pltpu.sync_copy(acc_sh, acc_hbm.at[cid])


@functools.cache
def _sc_kernels():
    mesh = plsc.VectorSubcoreMesh(core_axis_name="c", subcore_axis_name="s",
                                  num_cores=NC, num_subcores=NS)
    deg_fn = pl.kernel(
        _sc_deg_body,
        out_type=jax.ShapeDtypeStruct((NC, NPAD, D), jnp.float32),
        mesh=mesh,
        scratch_types=(
            pltpu.VMEM((SCH, CH), jnp.int32),
            pltpu.VMEM((CH, D), jnp.float32),
            pltpu.VMEM_SHARED((NPAD, D), jnp.float32),
            pltpu.SemaphoreType.DMA,
        ),
    )
    seg_fn = pl.kernel(
        _sc_seg_body,
        out_type=jax.ShapeDtypeStruct((NC, NPAD, D), jnp.float32),
        mesh=mesh,
        scratch_types=(
            pltpu.VMEM((SCH, CH), jnp.int32),
            pltpu.VMEM((SCH, CH), jnp.int32),
            pltpu.VMEM((CH, D), jnp.float32),
            pltpu.VMEM((CH, D), jnp.float32),
            pltpu.VMEM_SHARED((NPAD, D), jnp.float32),
            pltpu.SemaphoreType.DMA,
            pltpu.SemaphoreType.DMA,
        ),
    )
    return deg_fn, seg_fn


_BN = 1000
_GRID = N // _BN


def _enc_body(z_ref, nt_ref, emb_ref, w1_ref, b1_ref, w2_ref, b2_ref,
              src_ref, et_ref, x_ref, g_ref):
    zc = z_ref[...]
    oh = (lax.broadcasted_iota(jnp.int32, (_BN, 100), 1) == zc)
    zf = jnp.dot(oh.astype(jnp.float32), emb_ref[...],
                 preferred_element_type=jnp.float32)
    ntc = nt_ref[...]
    acc = jnp.zeros((_BN, D), jnp.float32)
    for t in range(NTYPES):
        h1 = jnp.maximum(
            jnp.dot(zf, w1_ref[t], preferred_element_type=jnp.float32)
            + b1_ref[t], 0.0)
        h2 = (jnp.dot(h1, w2_ref[t], preferred_element_type=jnp.float32)
              + b2_ref[t])
        acc = acc + jnp.where(ntc == t, 1.0, 0.0) * h2
    x_ref[...] = acc
    g_ref[...] = et_ref[...] * N + src_ref[...]


def _rel_body(x_ref, relw_ref, linw_ref, linb_ref, y_ref, zlin_ref):
    xb = x_ref[...]
    for r in range(R):
        y_ref[r] = jnp.dot(xb, relw_ref[r], preferred_element_type=jnp.float32)
    zlin_ref[...] = (jnp.dot(xb, linw_ref[...],
                             preferred_element_type=jnp.float32)
                     + linb_ref[...])


def _ln_body(zlin_ref, acc_ref, deg_ref, g_ref, b_ref, x_ref):
    out = acc_ref[0] + acc_ref[1]
    deg = jnp.maximum(deg_ref[0, :, 0:1] + deg_ref[1, :, 0:1], 1.0)
    t = zlin_ref[...] + out / deg
    mu = jnp.mean(t, axis=1, keepdims=True)
    var = jnp.mean((t - mu) ** 2, axis=1, keepdims=True)
    x_ref[...] = (t - mu) * lax.rsqrt(var + 1e-5) * g_ref[...] + b_ref[...]


def _pool_body(x_ref, w_ref, b_ref, o_ref):
    pooled = jnp.mean(x_ref[...], axis=0, keepdims=True)
    o_ref[...] = (jnp.dot(pooled, w_ref[...],
                          preferred_element_type=jnp.float32) + b_ref[...])


@functools.cache
def _tc_kernels(interpret=False):
    ee = E // 128
    encoder = pl.pallas_call(
        _enc_body,
        grid=(_GRID,),
        in_specs=[
            pl.BlockSpec((_BN, 1), lambda i: (i, 0)),
            pl.BlockSpec((_BN, 1), lambda i: (i, 0)),
            pl.BlockSpec((100, D), lambda i: (0, 0)),
            pl.BlockSpec((NTYPES, D, D), lambda i: (0, 0, 0)),
            pl.BlockSpec((NTYPES, 1, D), lambda i: (0, 0, 0)),
            pl.BlockSpec((NTYPES, D, D), lambda i: (0, 0, 0)),
            pl.BlockSpec((NTYPES, 1, D), lambda i: (0, 0, 0)),
            pl.BlockSpec((ee, 128), lambda i: (0, 0)),
            pl.BlockSpec((ee, 128), lambda i: (0, 0)),
        ],
        out_specs=[
            pl.BlockSpec((_BN, D), lambda i: (i, 0)),
            pl.BlockSpec((ee, 128), lambda i: (0, 0)),
        ],
        out_shape=[
            jax.ShapeDtypeStruct((N, D), jnp.float32),
            jax.ShapeDtypeStruct((ee, 128), jnp.int32),
        ],
        interpret=interpret,
    )
    relmm = pl.pallas_call(
        _rel_body,
        grid=(_GRID,),
        in_specs=[
            pl.BlockSpec((_BN, D), lambda i: (i, 0)),
            pl.BlockSpec((R, D, D), lambda i: (0, 0, 0)),
            pl.BlockSpec((D, D), lambda i: (0, 0)),
            pl.BlockSpec((1, D), lambda i: (0, 0)),
        ],
        out_specs=[
            pl.BlockSpec((R, _BN, D), lambda i: (0, i, 0)),
            pl.BlockSpec((_BN, D), lambda i: (i, 0)),
        ],
        out_shape=[
            jax.ShapeDtypeStruct((R, N, D), jnp.float32),
            jax.ShapeDtypeStruct((N, D), jnp.float32),
        ],
        interpret=interpret,
    )
    lnorm = pl.pallas_call(
        _ln_body,
        grid=(_GRID,),
        in_specs=[
            pl.BlockSpec((_BN, D), lambda i: (i, 0)),
            pl.BlockSpec((NC, _BN, D), lambda i: (0, i, 0)),
            pl.BlockSpec((NC, _BN, D), lambda i: (0, i, 0)),
            pl.BlockSpec((1, D), lambda i: (0, 0)),
            pl.BlockSpec((1, D), lambda i: (0, 0)),
        ],
        out_specs=pl.BlockSpec((_BN, D), lambda i: (i, 0)),
        out_shape=jax.ShapeDtypeStruct((N, D), jnp.float32),
        interpret=interpret,
    )
    pool = pl.pallas_call(
        _pool_body,
        out_shape=jax.ShapeDtypeStruct((1, 1), jnp.float32),
        interpret=interpret,
    )
    return encoder, relmm, lnorm, pool


@jax.jit
def kernel(z_embed, enc_W1, enc_b1, enc_W2, enc_b2, lin_W, lin_b, rel_W,
           ln_g, ln_b, reg_W, reg_b, z, node_type, edge_index, edge_type):
    encoder, relmm, lnorm, pool = _tc_kernels()
    deg_fn, seg_fn = _sc_kernels()

    src2 = edge_index[0].astype(jnp.int32).reshape(E // 128, 128)
    et2 = edge_type.astype(jnp.int32).reshape(E // 128, 128)
    dst3 = jnp.pad(edge_index[1].astype(jnp.int32).reshape(NW, EPW),
                   ((0, 0), (0, EPWP - EPW)),
                   constant_values=N).reshape(NW, NCH, CH)
    znd = jnp.zeros((NPAD, D), jnp.float32)

    x, g2 = encoder(z.astype(jnp.int32).reshape(N, 1),
                    node_type.astype(jnp.int32).reshape(N, 1),
                    z_embed, enc_W1, enc_b1.reshape(NTYPES, 1, D),
                    enc_W2, enc_b2.reshape(NTYPES, 1, D), src2, et2)
    g3 = jnp.pad(g2.reshape(NW, EPW),
                 ((0, 0), (0, EPWP - EPW))).reshape(NW, NCH, CH)

    deg2 = deg_fn(dst3, znd)

    for l in range(2):
        y, zlin = relmm(x, rel_W[l], lin_W[l], lin_b[l].reshape(1, D))
        acc2 = seg_fn(y.reshape(R * N, D), g3, dst3, znd)
        x = lnorm(zlin, acc2, deg2,
                  ln_g[l].reshape(1, D), ln_b[l].reshape(1, D))

    out = pool(x, reg_W, reg_b.reshape(1, 1))
    return out.reshape(1)


# speedup vs baseline: 1.3024x; 1.3024x over previous
import functools
import jax
import jax.numpy as jnp
from jax import lax
from jax.experimental import pallas as pl
from jax.experimental.pallas import tpu as pltpu
from jax.experimental.pallas import tpu_sc as plsc

N = 10000
E = 320000
D = 128
R = 3
NTYPES = 4

NC = 2
NS = 16
NW = NC * NS
EPW = E // NW
CH = 128
NCH = 80
EPWP = NCH * CH
SCH = 8
NSUP = NCH // SCH
NPAD = N + 8


def _worker():
    cid = lax.axis_index("c")
    sid = lax.axis_index("s")
    return cid, sid, sid * NC + cid


def _sc_deg_body(dst_hbm, znd_hbm, deg_hbm, dst_sup, ones_v, deg_sh, sem):
    cid, sid, wid = _worker()

    def fill(i, _):
        for j in range(D // 16):
            ones_v[i, pl.ds(j * 16, 16)] = jnp.full((16,), 1.0, jnp.float32)
        return 0
    lax.fori_loop(0, CH, fill, 0)

    @pl.when(sid == 0)
    def _():
        pltpu.sync_copy(znd_hbm, deg_sh)
    plsc.subcore_barrier()

    def sup(s, _):
        pltpu.sync_copy(dst_hbm.at[wid].at[pl.ds(s * SCH, SCH)], dst_sup)
        for j in range(SCH):
            pltpu.async_copy(ones_v, deg_sh.at[dst_sup.at[j]], sem, add=True)
        for j in range(SCH):
            pltpu.make_async_copy(ones_v, deg_sh.at[dst_sup.at[0]],
                                  sem).wait()
        return 0
    lax.fori_loop(0, NSUP, sup, 0)
    plsc.subcore_barrier()

    @pl.when(sid == 0)
    def _():
        pltpu.sync_copy(deg_sh, deg_hbm.at[cid])


CHS = 80
NCHS = EPW // CHS


def _sc_seg_body(y_hbm, g_hbm, dst_hbm, znd_hbm, acc_hbm,
                 g_v, dst_v, rows_v, acc_sh, sem):
    cid, sid, wid = _worker()

    @pl.when(sid == 0)
    def _():
        pltpu.sync_copy(znd_hbm, acc_sh)
    plsc.subcore_barrier()

    def chunk(c, _):
        base = wid * EPW + c * CHS
        pltpu.sync_copy(g_hbm.at[pl.ds(base, CHS)], g_v)
        pltpu.sync_copy(dst_hbm.at[pl.ds(base, CHS)], dst_v)
        pltpu.async_copy(y_hbm.at[g_v], rows_v, sem).wait()
        pltpu.sync_copy(rows_v, acc_sh.at[dst_v], add=True)
        return 0
    lax.fori_loop(0, NCHS, chunk, 0)
    plsc.subcore_barrier()

    @pl.when(sid == 0)
    def _():
        pltpu.sync_copy(acc_sh, acc_hbm.at[cid])


@functools.cache
def _sc_kernels():
    mesh = plsc.VectorSubcoreMesh(core_axis_name="c", subcore_axis_name="s",
                                  num_cores=NC, num_subcores=NS)
    deg_fn = pl.kernel(
        _sc_deg_body,
        out_type=jax.ShapeDtypeStruct((NC, NPAD, D), jnp.float32),
        mesh=mesh,
        scratch_types=(
            pltpu.VMEM((SCH, CH), jnp.int32),
            pltpu.VMEM((CH, D), jnp.float32),
            pltpu.VMEM_SHARED((NPAD, D), jnp.float32),
            pltpu.SemaphoreType.DMA,
        ),
    )
    seg_fn = pl.kernel(
        _sc_seg_body,
        out_type=jax.ShapeDtypeStruct((NC, NPAD, D), jnp.float32),
        mesh=mesh,
        scratch_types=(
            pltpu.VMEM((CHS,), jnp.int32),
            pltpu.VMEM((CHS,), jnp.int32),
            pltpu.VMEM((CHS, D), jnp.float32),
            pltpu.VMEM_SHARED((NPAD, D), jnp.float32),
            pltpu.SemaphoreType.DMA,
        ),
    )
    return deg_fn, seg_fn


_BN = 1000
_GRID = N // _BN


def _enc_body(z_ref, nt_ref, emb_ref, w1_ref, b1_ref, w2_ref, b2_ref,
              src_ref, et_ref, x_ref, g_ref):
    zc = z_ref[...]
    oh = (lax.broadcasted_iota(jnp.int32, (_BN, 100), 1) == zc)
    zf = jnp.dot(oh.astype(jnp.float32), emb_ref[...],
                 preferred_element_type=jnp.float32)
    ntc = nt_ref[...]
    acc = jnp.zeros((_BN, D), jnp.float32)
    for t in range(NTYPES):
        h1 = jnp.maximum(
            jnp.dot(zf, w1_ref[t], preferred_element_type=jnp.float32)
            + b1_ref[t], 0.0)
        h2 = (jnp.dot(h1, w2_ref[t], preferred_element_type=jnp.float32)
              + b2_ref[t])
        acc = acc + jnp.where(ntc == t, 1.0, 0.0) * h2
    x_ref[...] = acc
    g_ref[...] = et_ref[...] * N + src_ref[...]


def _rel_body(x_ref, relw_ref, linw_ref, linb_ref, y_ref, zlin_ref):
    xb = x_ref[...]
    for r in range(R):
        y_ref[r] = jnp.dot(xb, relw_ref[r], preferred_element_type=jnp.float32)
    zlin_ref[...] = (jnp.dot(xb, linw_ref[...],
                             preferred_element_type=jnp.float32)
                     + linb_ref[...])


def _ln_body(zlin_ref, acc_ref, deg_ref, g_ref, b_ref, x_ref):
    out = acc_ref[0] + acc_ref[1]
    deg = jnp.maximum(deg_ref[0, :, 0:1] + deg_ref[1, :, 0:1], 1.0)
    t = zlin_ref[...] + out / deg
    mu = jnp.mean(t, axis=1, keepdims=True)
    var = jnp.mean((t - mu) ** 2, axis=1, keepdims=True)
    x_ref[...] = (t - mu) * lax.rsqrt(var + 1e-5) * g_ref[...] + b_ref[...]


def _pool_body(x_ref, w_ref, b_ref, o_ref):
    pooled = jnp.mean(x_ref[...], axis=0, keepdims=True)
    o_ref[...] = (jnp.dot(pooled, w_ref[...],
                          preferred_element_type=jnp.float32) + b_ref[...])


@functools.cache
def _tc_kernels(interpret=False):
    ee = E // 128
    encoder = pl.pallas_call(
        _enc_body,
        grid=(_GRID,),
        in_specs=[
            pl.BlockSpec((_BN, 1), lambda i: (i, 0)),
            pl.BlockSpec((_BN, 1), lambda i: (i, 0)),
            pl.BlockSpec((100, D), lambda i: (0, 0)),
            pl.BlockSpec((NTYPES, D, D), lambda i: (0, 0, 0)),
            pl.BlockSpec((NTYPES, 1, D), lambda i: (0, 0, 0)),
            pl.BlockSpec((NTYPES, D, D), lambda i: (0, 0, 0)),
            pl.BlockSpec((NTYPES, 1, D), lambda i: (0, 0, 0)),
            pl.BlockSpec((ee, 128), lambda i: (0, 0)),
            pl.BlockSpec((ee, 128), lambda i: (0, 0)),
        ],
        out_specs=[
            pl.BlockSpec((_BN, D), lambda i: (i, 0)),
            pl.BlockSpec((ee, 128), lambda i: (0, 0)),
        ],
        out_shape=[
            jax.ShapeDtypeStruct((N, D), jnp.float32),
            jax.ShapeDtypeStruct((ee, 128), jnp.int32),
        ],
        interpret=interpret,
    )
    relmm = pl.pallas_call(
        _rel_body,
        grid=(_GRID,),
        in_specs=[
            pl.BlockSpec((_BN, D), lambda i: (i, 0)),
            pl.BlockSpec((R, D, D), lambda i: (0, 0, 0)),
            pl.BlockSpec((D, D), lambda i: (0, 0)),
            pl.BlockSpec((1, D), lambda i: (0, 0)),
        ],
        out_specs=[
            pl.BlockSpec((R, _BN, D), lambda i: (0, i, 0)),
            pl.BlockSpec((_BN, D), lambda i: (i, 0)),
        ],
        out_shape=[
            jax.ShapeDtypeStruct((R, N, D), jnp.float32),
            jax.ShapeDtypeStruct((N, D), jnp.float32),
        ],
        interpret=interpret,
    )
    lnorm = pl.pallas_call(
        _ln_body,
        grid=(_GRID,),
        in_specs=[
            pl.BlockSpec((_BN, D), lambda i: (i, 0)),
            pl.BlockSpec((NC, _BN, D), lambda i: (0, i, 0)),
            pl.BlockSpec((NC, _BN, D), lambda i: (0, i, 0)),
            pl.BlockSpec((1, D), lambda i: (0, 0)),
            pl.BlockSpec((1, D), lambda i: (0, 0)),
        ],
        out_specs=pl.BlockSpec((_BN, D), lambda i: (i, 0)),
        out_shape=jax.ShapeDtypeStruct((N, D), jnp.float32),
        interpret=interpret,
    )
    pool = pl.pallas_call(
        _pool_body,
        out_shape=jax.ShapeDtypeStruct((1, 1), jnp.float32),
        interpret=interpret,
    )
    return encoder, relmm, lnorm, pool


@jax.jit
def kernel(z_embed, enc_W1, enc_b1, enc_W2, enc_b2, lin_W, lin_b, rel_W,
           ln_g, ln_b, reg_W, reg_b, z, node_type, edge_index, edge_type):
    encoder, relmm, lnorm, pool = _tc_kernels()
    deg_fn, seg_fn = _sc_kernels()

    src2 = edge_index[0].astype(jnp.int32).reshape(E // 128, 128)
    et2 = edge_type.astype(jnp.int32).reshape(E // 128, 128)
    dst3 = jnp.pad(edge_index[1].astype(jnp.int32).reshape(NW, EPW),
                   ((0, 0), (0, EPWP - EPW)),
                   constant_values=N).reshape(NW, NCH, CH)
    znd = jnp.zeros((NPAD, D), jnp.float32)

    x, g2 = encoder(z.astype(jnp.int32).reshape(N, 1),
                    node_type.astype(jnp.int32).reshape(N, 1),
                    z_embed, enc_W1, enc_b1.reshape(NTYPES, 1, D),
                    enc_W2, enc_b2.reshape(NTYPES, 1, D), src2, et2)
    deg2 = deg_fn(dst3, znd)

    gflat = g2.reshape(E)
    dstflat = edge_index[1].astype(jnp.int32)
    for l in range(2):
        y, zlin = relmm(x, rel_W[l], lin_W[l], lin_b[l].reshape(1, D))
        acc2 = seg_fn(y.reshape(R * N, D), gflat, dstflat, znd)
        x = lnorm(zlin, acc2, deg2,
                  ln_g[l].reshape(1, D), ln_b[l].reshape(1, D))

    out = pool(x, reg_W, reg_b.reshape(1, 1))
    return out.reshape(1)


# speedup vs baseline: 1.9025x; 1.4608x over previous
import functools
import jax
import jax.numpy as jnp
from jax import lax
from jax.experimental import pallas as pl
from jax.experimental.pallas import tpu as pltpu
from jax.experimental.pallas import tpu_sc as plsc

N = 10000
E = 320000
D = 128
R = 3
NTYPES = 4

NC = 2
NS = 16
NW = NC * NS
EPW = E // NW
CH = 128
NCH = 80
EPWP = NCH * CH
SCH = 8
NSUP = NCH // SCH
NPAD = N + 8


def _worker():
    cid = lax.axis_index("c")
    sid = lax.axis_index("s")
    return cid, sid, sid * NC + cid


def _sc_deg_body(dst_hbm, znd_hbm, deg_hbm, dst_sup, ones_v, deg_sh, sem):
    cid, sid, wid = _worker()

    def fill(i, _):
        for j in range(D // 16):
            ones_v[i, pl.ds(j * 16, 16)] = jnp.full((16,), 1.0, jnp.float32)
        return 0
    lax.fori_loop(0, CH, fill, 0)

    @pl.when(sid == 0)
    def _():
        pltpu.sync_copy(znd_hbm, deg_sh)
    plsc.subcore_barrier()

    def sup(s, _):
        pltpu.sync_copy(dst_hbm.at[wid].at[pl.ds(s * SCH, SCH)], dst_sup)
        for j in range(SCH):
            pltpu.async_copy(ones_v, deg_sh.at[dst_sup.at[j]], sem, add=True)
        for j in range(SCH):
            pltpu.make_async_copy(ones_v, deg_sh.at[dst_sup.at[0]],
                                  sem).wait()
        return 0
    lax.fori_loop(0, NSUP, sup, 0)
    plsc.subcore_barrier()

    @pl.when(sid == 0)
    def _():
        pltpu.sync_copy(deg_sh, deg_hbm.at[cid])


CHS = 80
NCHS = EPW // CHS


RING = 4
NSUPS = NCHS // RING
TAILS = NCHS - NSUPS * RING


def _sc_seg_body(y_hbm, g_hbm, dst_hbm, znd_hbm, acc_hbm,
                 g0, g1, g2, g3, d0, d1, d2, d3, r0, r1, r2, r3,
                 acc_sh, sg0, sg1, sg2, sg3, ss):
    cid, sid, wid = _worker()
    gs = (g0, g1, g2, g3)
    ds = (d0, d1, d2, d3)
    rs = (r0, r1, r2, r3)
    sgs = (sg0, sg1, sg2, sg3)

    @pl.when(sid == 0)
    def _():
        pltpu.sync_copy(znd_hbm, acc_sh)
    plsc.subcore_barrier()

    def sup(s, _):
        for j in range(RING):
            base = wid * EPW + (s * RING + j) * CHS
            pltpu.sync_copy(g_hbm.at[pl.ds(base, CHS)], gs[j])
            pltpu.sync_copy(dst_hbm.at[pl.ds(base, CHS)], ds[j])
            pltpu.async_copy(y_hbm.at[gs[j]], rs[j], sgs[j])
        for j in range(RING):
            pltpu.make_async_copy(y_hbm.at[gs[j]], rs[j], sgs[j]).wait()
            pltpu.async_copy(rs[j], acc_sh.at[ds[j]], ss, add=True)
        for j in range(RING):
            pltpu.make_async_copy(rs[0], acc_sh.at[ds[0]], ss).wait()
        return 0
    lax.fori_loop(0, NSUPS, sup, 0)
    for t in range(TAILS):
        base = wid * EPW + (NSUPS * RING + t) * CHS
        pltpu.sync_copy(g_hbm.at[pl.ds(base, CHS)], g0)
        pltpu.sync_copy(dst_hbm.at[pl.ds(base, CHS)], d0)
        pltpu.async_copy(y_hbm.at[g0], r0, sg0).wait()
        pltpu.sync_copy(r0, acc_sh.at[d0], add=True)
    plsc.subcore_barrier()

    @pl.when(sid == 0)
    def _():
        pltpu.sync_copy(acc_sh, acc_hbm.at[cid])


@functools.cache
def _sc_kernels():
    mesh = plsc.VectorSubcoreMesh(core_axis_name="c", subcore_axis_name="s",
                                  num_cores=NC, num_subcores=NS)
    deg_fn = pl.kernel(
        _sc_deg_body,
        out_type=jax.ShapeDtypeStruct((NC, NPAD, D), jnp.float32),
        mesh=mesh,
        scratch_types=(
            pltpu.VMEM((SCH, CH), jnp.int32),
            pltpu.VMEM((CH, D), jnp.float32),
            pltpu.VMEM_SHARED((NPAD, D), jnp.float32),
            pltpu.SemaphoreType.DMA,
        ),
    )
    seg_fn = pl.kernel(
        _sc_seg_body,
        out_type=jax.ShapeDtypeStruct((NC, NPAD, D), jnp.float32),
        mesh=mesh,
        scratch_types=(
            pltpu.VMEM((CHS,), jnp.int32),
            pltpu.VMEM((CHS,), jnp.int32),
            pltpu.VMEM((CHS,), jnp.int32),
            pltpu.VMEM((CHS,), jnp.int32),
            pltpu.VMEM((CHS,), jnp.int32),
            pltpu.VMEM((CHS,), jnp.int32),
            pltpu.VMEM((CHS,), jnp.int32),
            pltpu.VMEM((CHS,), jnp.int32),
            pltpu.VMEM((CHS, D), jnp.float32),
            pltpu.VMEM((CHS, D), jnp.float32),
            pltpu.VMEM((CHS, D), jnp.float32),
            pltpu.VMEM((CHS, D), jnp.float32),
            pltpu.VMEM_SHARED((NPAD, D), jnp.float32),
            pltpu.SemaphoreType.DMA,
            pltpu.SemaphoreType.DMA,
            pltpu.SemaphoreType.DMA,
            pltpu.SemaphoreType.DMA,
            pltpu.SemaphoreType.DMA,
        ),
    )
    return deg_fn, seg_fn


_BN = 1000
_GRID = N // _BN


def _enc_body(z_ref, nt_ref, emb_ref, w1_ref, b1_ref, w2_ref, b2_ref,
              src_ref, et_ref, x_ref, g_ref):
    zc = z_ref[...]
    oh = (lax.broadcasted_iota(jnp.int32, (_BN, 100), 1) == zc)
    zf = jnp.dot(oh.astype(jnp.float32), emb_ref[...],
                 preferred_element_type=jnp.float32)
    ntc = nt_ref[...]
    acc = jnp.zeros((_BN, D), jnp.float32)
    for t in range(NTYPES):
        h1 = jnp.maximum(
            jnp.dot(zf, w1_ref[t], preferred_element_type=jnp.float32)
            + b1_ref[t], 0.0)
        h2 = (jnp.dot(h1, w2_ref[t], preferred_element_type=jnp.float32)
              + b2_ref[t])
        acc = acc + jnp.where(ntc == t, 1.0, 0.0) * h2
    x_ref[...] = acc
    g_ref[...] = et_ref[...] * N + src_ref[...]


def _rel_body(x_ref, relw_ref, linw_ref, linb_ref, y_ref, zlin_ref):
    xb = x_ref[...]
    for r in range(R):
        y_ref[r] = jnp.dot(xb, relw_ref[r], preferred_element_type=jnp.float32)
    zlin_ref[...] = (jnp.dot(xb, linw_ref[...],
                             preferred_element_type=jnp.float32)
                     + linb_ref[...])


def _ln_body(zlin_ref, acc_ref, deg_ref, g_ref, b_ref, x_ref):
    out = acc_ref[0] + acc_ref[1]
    deg = jnp.maximum(deg_ref[0, :, 0:1] + deg_ref[1, :, 0:1], 1.0)
    t = zlin_ref[...] + out / deg
    mu = jnp.mean(t, axis=1, keepdims=True)
    var = jnp.mean((t - mu) ** 2, axis=1, keepdims=True)
    x_ref[...] = (t - mu) * lax.rsqrt(var + 1e-5) * g_ref[...] + b_ref[...]


def _pool_body(x_ref, w_ref, b_ref, o_ref):
    pooled = jnp.mean(x_ref[...], axis=0, keepdims=True)
    o_ref[...] = (jnp.dot(pooled, w_ref[...],
                          preferred_element_type=jnp.float32) + b_ref[...])


@functools.cache
def _tc_kernels(interpret=False):
    ee = E // 128
    encoder = pl.pallas_call(
        _enc_body,
        grid=(_GRID,),
        in_specs=[
            pl.BlockSpec((_BN, 1), lambda i: (i, 0)),
            pl.BlockSpec((_BN, 1), lambda i: (i, 0)),
            pl.BlockSpec((100, D), lambda i: (0, 0)),
            pl.BlockSpec((NTYPES, D, D), lambda i: (0, 0, 0)),
            pl.BlockSpec((NTYPES, 1, D), lambda i: (0, 0, 0)),
            pl.BlockSpec((NTYPES, D, D), lambda i: (0, 0, 0)),
            pl.BlockSpec((NTYPES, 1, D), lambda i: (0, 0, 0)),
            pl.BlockSpec((ee, 128), lambda i: (0, 0)),
            pl.BlockSpec((ee, 128), lambda i: (0, 0)),
        ],
        out_specs=[
            pl.BlockSpec((_BN, D), lambda i: (i, 0)),
            pl.BlockSpec((ee, 128), lambda i: (0, 0)),
        ],
        out_shape=[
            jax.ShapeDtypeStruct((N, D), jnp.float32),
            jax.ShapeDtypeStruct((ee, 128), jnp.int32),
        ],
        interpret=interpret,
    )
    relmm = pl.pallas_call(
        _rel_body,
        grid=(_GRID,),
        in_specs=[
            pl.BlockSpec((_BN, D), lambda i: (i, 0)),
            pl.BlockSpec((R, D, D), lambda i: (0, 0, 0)),
            pl.BlockSpec((D, D), lambda i: (0, 0)),
            pl.BlockSpec((1, D), lambda i: (0, 0)),
        ],
        out_specs=[
            pl.BlockSpec((R, _BN, D), lambda i: (0, i, 0)),
            pl.BlockSpec((_BN, D), lambda i: (i, 0)),
        ],
        out_shape=[
            jax.ShapeDtypeStruct((R, N, D), jnp.float32),
            jax.ShapeDtypeStruct((N, D), jnp.float32),
        ],
        interpret=interpret,
    )
    lnorm = pl.pallas_call(
        _ln_body,
        grid=(_GRID,),
        in_specs=[
            pl.BlockSpec((_BN, D), lambda i: (i, 0)),
            pl.BlockSpec((NC, _BN, D), lambda i: (0, i, 0)),
            pl.BlockSpec((NC, _BN, D), lambda i: (0, i, 0)),
            pl.BlockSpec((1, D), lambda i: (0, 0)),
            pl.BlockSpec((1, D), lambda i: (0, 0)),
        ],
        out_specs=pl.BlockSpec((_BN, D), lambda i: (i, 0)),
        out_shape=jax.ShapeDtypeStruct((N, D), jnp.float32),
        interpret=interpret,
    )
    pool = pl.pallas_call(
        _pool_body,
        out_shape=jax.ShapeDtypeStruct((1, 1), jnp.float32),
        interpret=interpret,
    )
    return encoder, relmm, lnorm, pool


@jax.jit
def kernel(z_embed, enc_W1, enc_b1, enc_W2, enc_b2, lin_W, lin_b, rel_W,
           ln_g, ln_b, reg_W, reg_b, z, node_type, edge_index, edge_type):
    encoder, relmm, lnorm, pool = _tc_kernels()
    deg_fn, seg_fn = _sc_kernels()

    src2 = edge_index[0].astype(jnp.int32).reshape(E // 128, 128)
    et2 = edge_type.astype(jnp.int32).reshape(E // 128, 128)
    dst3 = jnp.pad(edge_index[1].astype(jnp.int32).reshape(NW, EPW),
                   ((0, 0), (0, EPWP - EPW)),
                   constant_values=N).reshape(NW, NCH, CH)
    znd = jnp.zeros((NPAD, D), jnp.float32)

    x, g2 = encoder(z.astype(jnp.int32).reshape(N, 1),
                    node_type.astype(jnp.int32).reshape(N, 1),
                    z_embed, enc_W1, enc_b1.reshape(NTYPES, 1, D),
                    enc_W2, enc_b2.reshape(NTYPES, 1, D), src2, et2)
    deg2 = deg_fn(dst3, znd)

    gflat = g2.reshape(E)
    dstflat = edge_index[1].astype(jnp.int32)
    for l in range(2):
        y, zlin = relmm(x, rel_W[l], lin_W[l], lin_b[l].reshape(1, D))
        acc2 = seg_fn(y.reshape(R * N, D), gflat, dstflat, znd)
        x = lnorm(zlin, acc2, deg2,
                  ln_g[l].reshape(1, D), ln_b[l].reshape(1, D))

    out = pool(x, reg_W, reg_b.reshape(1, 1))
    return out.reshape(1)
